# baseline (speedup 1.0000x reference)
import jax
import jax.numpy as jnp
from jax.experimental import pallas as pl
from jax.experimental.pallas import tpu as pltpu
from jax.experimental.pallas import tpu_sc as plsc

N = 32768
C = 64
M = 1024
BLK = 1024
GW = 256


def _vq_kernel(q_ref, ktn_ref, idx_ref, oh_ref, cnt_ref, perp_ref):
    i = pl.program_id(0)
    nblocks = pl.num_programs(0)

    qb = q_ref[...]
    ktn = ktn_ref[...]

    l2q = jnp.sum(qb * qb, axis=1, keepdims=True)
    l2k = 0.25 * jnp.sum(ktn * ktn, axis=0, keepdims=True)
    simn = jnp.dot(qb, ktn, preferred_element_type=jnp.float32)
    dist = (l2q + l2k) + simn

    mval = jnp.min(dist, axis=1, keepdims=True)
    lane = jax.lax.broadcasted_iota(jnp.int32, dist.shape, 1)
    idx = jnp.min(jnp.where(dist == mval, lane, M), axis=1, keepdims=True)

    onehot = (lane == idx).astype(jnp.float32)
    oh_ref[...] = onehot
    idx_ref[...] = idx

    part = jnp.sum(onehot, axis=0, keepdims=True)

    @pl.when(i == 0)
    def _init():
        cnt_ref[...] = part

    @pl.when(i != 0)
    def _acc():
        cnt_ref[...] += part

    @pl.when(i == nblocks - 1)
    def _finish():
        p = cnt_ref[...] * (1.0 / N)
        s = jnp.sum(p * jnp.log(p + 1e-10), axis=1, keepdims=True)
        perp_ref[...] = jnp.exp(-s)


def _sc_gather_z(kp, idx_row):
    mesh = plsc.VectorSubcoreMesh(core_axis_name="c", subcore_axis_name="s")

    @pl.kernel(out_type=jax.ShapeDtypeStruct((N, 128), jnp.float32), mesh=mesh)
    def kern(k_hbm, i_hbm, o_hbm):
        def body(i_vmem, o_vmem):
            pltpu.sync_copy(k_hbm.at[i_vmem.at[0]], o_vmem)

        pltpu.emit_pipeline(
            body,
            grid=(N // GW,),
            in_specs=[pl.BlockSpec((1, GW), lambda i: (0, i))],
            out_specs=[pl.BlockSpec((GW, 128), lambda i: (i, 0))],
            core_axis_name=("c", "s"),
            dimension_semantics=(pltpu.PARALLEL,),
        )(i_hbm, o_hbm)

    return kern(kp, idx_row)


@jax.jit
def kernel(q, k):
    ktn = k.T * (-2.0)
    grid = (N // BLK,)
    idx, onehot, _cnt, perp = pl.pallas_call(
        _vq_kernel,
        grid=grid,
        in_specs=[
            pl.BlockSpec((BLK, C), lambda i: (i, 0)),
            pl.BlockSpec((C, M), lambda i: (0, 0)),
        ],
        out_specs=[
            pl.BlockSpec((BLK, 1), lambda i: (i, 0)),
            pl.BlockSpec((BLK, M), lambda i: (i, 0)),
            pl.BlockSpec((1, M), lambda i: (0, 0)),
            pl.BlockSpec((1, 1), lambda i: (0, 0)),
        ],
        out_shape=[
            jax.ShapeDtypeStruct((N, 1), jnp.int32),
            jax.ShapeDtypeStruct((N, M), jnp.float32),
            jax.ShapeDtypeStruct((1, M), jnp.float32),
            jax.ShapeDtypeStruct((1, 1), jnp.float32),
        ],
        compiler_params=pltpu.CompilerParams(
            dimension_semantics=("arbitrary",),
        ),
    )(q, ktn)
    kp = jnp.pad(k, ((0, 0), (0, 128 - C)))
    zw = _sc_gather_z(kp, idx.reshape(1, N))
    return (zw[:, :C], idx.reshape(N), onehot, perp[0, 0])


# speedup vs baseline: 1.3540x; 1.3540x over previous
import jax
import jax.numpy as jnp
from jax.experimental import pallas as pl
from jax.experimental.pallas import tpu as pltpu

N = 32768
C = 64
M = 1024
BLK = 2048


def _vq_kernel(q_ref, k_ref, ktn_ref, z_ref, idx_ref, oh_ref, cnt_ref, perp_ref):
    i = pl.program_id(0)
    nblocks = pl.num_programs(0)

    qb = q_ref[...]
    ktn = ktn_ref[...]

    l2q = jnp.sum(qb * qb, axis=1, keepdims=True)
    l2k = 0.25 * jnp.sum(ktn * ktn, axis=0, keepdims=True)
    simn = jnp.dot(qb, ktn, preferred_element_type=jnp.float32)
    dist = (l2q + l2k) + simn

    mval = jnp.min(dist, axis=1, keepdims=True)
    lane = jax.lax.broadcasted_iota(jnp.int32, dist.shape, 1)
    idx = jnp.min(jnp.where(dist == mval, lane, M), axis=1, keepdims=True)

    onehot = (lane == idx).astype(jnp.float32)
    oh_ref[...] = onehot
    idx_ref[...] = idx
    z_ref[...] = jnp.dot(onehot, k_ref[...], preferred_element_type=jnp.float32)

    part = jnp.sum(onehot, axis=0, keepdims=True)

    @pl.when(i == 0)
    def _init():
        cnt_ref[...] = part

    @pl.when(i != 0)
    def _acc():
        cnt_ref[...] += part

    @pl.when(i == nblocks - 1)
    def _finish():
        p = cnt_ref[...] * (1.0 / N)
        s = jnp.sum(p * jnp.log(p + 1e-10), axis=1, keepdims=True)
        perp_ref[...] = jnp.exp(-s)


@jax.jit
def kernel(q, k):
    ktn = k.T * (-2.0)
    grid = (N // BLK,)
    z, idx, onehot, _cnt, perp = pl.pallas_call(
        _vq_kernel,
        grid=grid,
        in_specs=[
            pl.BlockSpec((BLK, C), lambda i: (i, 0)),
            pl.BlockSpec((M, C), lambda i: (0, 0)),
            pl.BlockSpec((C, M), lambda i: (0, 0)),
        ],
        out_specs=[
            pl.BlockSpec((BLK, C), lambda i: (i, 0)),
            pl.BlockSpec((BLK, 1), lambda i: (i, 0)),
            pl.BlockSpec((BLK, M), lambda i: (i, 0)),
            pl.BlockSpec((1, M), lambda i: (0, 0)),
            pl.BlockSpec((1, 1), lambda i: (0, 0)),
        ],
        out_shape=[
            jax.ShapeDtypeStruct((N, C), jnp.float32),
            jax.ShapeDtypeStruct((N, 1), jnp.int32),
            jax.ShapeDtypeStruct((N, M), jnp.float32),
            jax.ShapeDtypeStruct((1, M), jnp.float32),
            jax.ShapeDtypeStruct((1, 1), jnp.float32),
        ],
        compiler_params=pltpu.CompilerParams(
            dimension_semantics=("arbitrary",),
        ),
    )(q, k, ktn)
    return (z, idx.reshape(N), onehot, perp[0, 0])
